# baseline (speedup 1.0000x reference)
import jax
import jax.numpy as jnp
from jax.experimental import pallas as pl
from jax.experimental.pallas import tpu as pltpu

B = 8
N = 1024
S1 = 512
K1 = 32
R1SQ = 0.2 * 0.2
S2 = 128
K2 = 64
R2SQ = 0.4 * 0.4
G1 = 8
G2 = 8

_F32 = jnp.float32
_NEG_INF = float('-inf')
_POS_INF = float('inf')


def _mm(a, b):
    return jnp.dot(a, b, preferred_element_type=jnp.float32)


def _tree(parts, op):
    while len(parts) > 1:
        nxt = [op(parts[i], parts[i + 1]) for i in range(0, len(parts) - 1, 2)]
        if len(parts) % 2:
            nxt.append(parts[-1])
        parts = nxt
    return parts[0]


def _row_reduce(x, op, jop):
    n = x.shape[1]
    parts = [x[:, k * 128:(k + 1) * 128] for k in range(n // 128)]
    return jop(_tree(parts, op), axis=1, keepdims=True)


def _fps_level(pts, S):
    px, py, pz = pts
    n = px.shape[1]
    nb = n // 128
    ii = jax.lax.broadcasted_iota(jnp.int32, (B, n), 1)
    iiS = jax.lax.broadcasted_iota(jnp.int32, (B, S), 1)

    def blk(v):
        return [v[:, k * 128:(k + 1) * 128] for k in range(nb)]

    ii_b = blk(ii)
    px_b = blk(px)
    py_b = blk(py)
    pz_b = blk(pz)

    def argmax_tree(mind):
        parts = list(zip(blk(mind), ii_b, px_b, py_b, pz_b))
        while len(parts) > 1:
            nxt = []
            for k in range(0, len(parts) - 1, 2):
                va, ia, xa, ya, za = parts[k]
                vb, ib, xb, yb, zb = parts[k + 1]
                take = (va > vb) | ((va == vb) & (ia < ib))
                nxt.append((jnp.where(take, va, vb),
                            jnp.where(take, ia, ib),
                            jnp.where(take, xa, xb),
                            jnp.where(take, ya, yb),
                            jnp.where(take, za, zb)))
            if len(parts) % 2:
                nxt.append(parts[-1])
            parts = nxt
        return parts[0]

    x0 = px[:, 0:1]
    y0 = py[:, 0:1]
    z0 = pz[:, 0:1]
    mind = ((px - x0) * (px - x0) + (py - y0) * (py - y0)
            + (pz - z0) * (pz - z0))
    z = _F32(0.0)
    rowx = jnp.where(iiS == 0, x0, z)
    rowy = jnp.where(iiS == 0, y0, z)
    rowz = jnp.where(iiS == 0, z0, z)

    def body(i, carry):
        mind, rowx, rowy, rowz = carry
        v, a, X, Y, Z = argmax_tree(mind)
        mx = jnp.max(v, axis=1, keepdims=True)
        nxt = jnp.min(jnp.where(v == mx, a, jnp.int32(n)),
                      axis=1, keepdims=True)
        sel = (v == mx) & (a == nxt)
        sx = jnp.sum(jnp.where(sel, X, z), axis=1, keepdims=True)
        sy = jnp.sum(jnp.where(sel, Y, z), axis=1, keepdims=True)
        sz = jnp.sum(jnp.where(sel, Z, z), axis=1, keepdims=True)
        ohS = iiS == i
        rowx = rowx + jnp.where(ohS, sx, z)
        rowy = rowy + jnp.where(ohS, sy, z)
        rowz = rowz + jnp.where(ohS, sz, z)
        d = ((px - sx) * (px - sx) + (py - sy) * (py - sy)
             + (pz - sz) * (pz - sz))
        return jnp.minimum(mind, d), rowx, rowy, rowz

    _, rowx, rowy, rowz = jax.lax.fori_loop(
        1, S, body, (mind, rowx, rowy, rowz), unroll=4)
    return rowx, rowy, rowz


def _fps_body(posT_ref, r1_ref, r2_ref):
    pT = posT_ref[:, :, :]
    px = pT[:, 0, :]
    py = pT[:, 1, :]
    pz = pT[:, 2, :]
    r1x, r1y, r1z = _fps_level((px, py, pz), S1)
    r1_ref[:, 0, :] = r1x
    r1_ref[:, 1, :] = r1y
    r1_ref[:, 2, :] = r1z
    r2x, r2y, r2z = _fps_level((r1x, r1y, r1z), S2)
    r2_ref[:, 0, :] = r2x
    r2_ref[:, 1, :] = r2y
    r2_ref[:, 2, :] = r2z


def _sa_body(pos3_ref, posT_ref, c1T_ref, r1_ref, c2T_ref,
             w11, b11, w12, b12, w13, b13,
             w21, b21, w22, b22, w23, b23,
             w31, b31, w32, b32, w33, b33,
             fw1, fb1, bg1, bb1, fw2, fb2, bg2, bb2, cw, cb,
             out_ref):
    p3 = pos3_ref[0]
    pT = posT_ref[0]
    px_row = pT[0:1, :]
    py_row = pT[1:2, :]
    pz_row = pT[2:3, :]
    c1 = c1T_ref[0]
    colx1 = c1[:, 0:1]
    coly1 = c1[:, 1:2]
    colz1 = c1[:, 2:3]
    r1 = r1_ref[0]
    r1x = r1[0:1, :]
    r1y = r1[1:2, :]
    r1z = r1[2:3, :]
    c2 = c2T_ref[0]
    colx2 = c2[:, 0:1]
    coly2 = c2[:, 1:2]
    colz2 = c2[:, 2:3]

    dx = colx1 - px_row
    dy = coly1 - py_row
    dz = colz1 - pz_row
    d2a = dx * dx + dy * dy + dz * dz
    jj1 = jax.lax.broadcasted_iota(jnp.int32, (S1, N), 1)

    def sa1_body(j, carry):
        d2m, hmax = carry
        rels = []
        keeps = []
        for _ in range(G1):
            m = jnp.min(d2m, axis=1, keepdims=True)
            nidx = jnp.min(jnp.where(d2m == m, jj1, jnp.int32(2 * N)),
                           axis=1, keepdims=True)
            oh = jj1 == nidx
            rels.append(_mm(oh.astype(jnp.float32), p3) - c1)
            keeps.append(m <= _F32(R1SQ))
            d2m = jnp.where(oh, _POS_INF, d2m)
        rel = jnp.concatenate(rels, axis=0)
        keep = jnp.concatenate(keeps, axis=0)
        a = jnp.maximum(_mm(rel, w11[:, :]) + b11[:, :], _F32(0.0))
        a = jnp.maximum(_mm(a, w12[:, :]) + b12[:, :], _F32(0.0))
        msg = jnp.maximum(_mm(a, w13[:, :]) + b13[:, :], _F32(0.0))
        msg = jnp.where(keep, msg, _NEG_INF)
        for s in range(G1):
            hmax = jnp.maximum(hmax, msg[s * S1:(s + 1) * S1, :])
        return d2m, hmax

    _, h1 = jax.lax.fori_loop(
        0, K1 // G1, sa1_body,
        (d2a, jnp.full((S1, 128), _NEG_INF, jnp.float32)))

    dx2 = colx2 - r1x
    dy2 = coly2 - r1y
    dz2 = colz2 - r1z
    d2b = dx2 * dx2 + dy2 * dy2 + dz2 * dz2
    jj2 = jax.lax.broadcasted_iota(jnp.int32, (S2, S1), 1)
    hp1 = jnp.concatenate([h1, c1], axis=1)

    def sa2_body(j, carry):
        d2m, hmax = carry
        gs = []
        keeps = []
        for _ in range(G2):
            m = jnp.min(d2m, axis=1, keepdims=True)
            nidx = jnp.min(jnp.where(d2m == m, jj2, jnp.int32(2 * S1)),
                           axis=1, keepdims=True)
            oh = jj2 == nidx
            gs.append(_mm(oh.astype(jnp.float32), hp1))
            keeps.append(m <= _F32(R2SQ))
            d2m = jnp.where(oh, _POS_INF, d2m)
        g = jnp.concatenate(gs, axis=0)
        rel3 = g[:, 128:131] - jnp.concatenate([c2] * G2, axis=0)
        keep = jnp.concatenate(keeps, axis=0)
        a = (_mm(g[:, 0:128], w21[0:128, :]) + _mm(rel3, w21[128:131, :])
             + b21[:, :])
        a = jnp.maximum(a, _F32(0.0))
        a = jnp.maximum(_mm(a, w22[:, :]) + b22[:, :], _F32(0.0))
        msg = jnp.maximum(_mm(a, w23[:, :]) + b23[:, :], _F32(0.0))
        msg = jnp.where(keep, msg, _NEG_INF)
        for s in range(G2):
            hmax = jnp.maximum(hmax, msg[s * S2:(s + 1) * S2, :])
        return d2m, hmax

    _, h2 = jax.lax.fori_loop(
        0, K2 // G2, sa2_body,
        (d2b, jnp.full((S2, 256), _NEG_INF, jnp.float32)))

    a = _mm(h2, w31[0:256, :]) + _mm(c2, w31[256:259, :]) + b31[:, :]
    a = jnp.maximum(a, _F32(0.0))
    a = jnp.maximum(_mm(a, w32[:, :]) + b32[:, :], _F32(0.0))
    a = jnp.maximum(_mm(a, w33[:, :]) + b33[:, :], _F32(0.0))
    g = jnp.max(a, axis=0, keepdims=True)

    inv = _F32(1.0) / jnp.sqrt(_F32(1.0 + 1e-5))
    x = _mm(g, fw1[:, :]) + fb1[:, :]
    x = jnp.maximum(x * inv * bg1[:, :] + bb1[:, :], _F32(0.0))
    x = _mm(x, fw2[:, :]) + fb2[:, :]
    x = jnp.maximum(x * inv * bg2[:, :] + bb2[:, :], _F32(0.0))
    out_ref[0] = _mm(x, cw[:, :]) + cb[:, :]


def kernel(pos, batch, params):
    del batch
    pos3 = pos.reshape(B, N, 3)
    posT = pos3.transpose(0, 2, 1)

    rows1, rows2 = pl.pallas_call(
        _fps_body,
        grid=(1,),
        in_specs=[pl.BlockSpec((B, 3, N), lambda i: (0, 0, 0))],
        out_specs=[
            pl.BlockSpec((B, 3, S1), lambda i: (0, 0, 0)),
            pl.BlockSpec((B, 3, S2), lambda i: (0, 0, 0)),
        ],
        out_shape=[
            jax.ShapeDtypeStruct((B, 3, S1), jnp.float32),
            jax.ShapeDtypeStruct((B, 3, S2), jnp.float32),
        ],
    )(posT)
    c1T = rows1.transpose(0, 2, 1)
    c2T = rows2.transpose(0, 2, 1)

    def row(v):
        return v.reshape(1, -1)

    flat = []
    for name in ('sa1', 'sa2', 'sa3'):
        for W, b in params[name]:
            flat += [W, row(b)]
    flat += [params['fc1'][0], row(params['fc1'][1]),
             row(params['bn1'][0]), row(params['bn1'][1]),
             params['fc2'][0], row(params['fc2'][1]),
             row(params['bn2'][0]), row(params['bn2'][1]),
             params['cls'][0], row(params['cls'][1])]
    full = [pl.BlockSpec(w.shape, lambda i, nd=w.ndim: (0,) * nd)
            for w in flat]

    out = pl.pallas_call(
        _sa_body,
        grid=(B,),
        in_specs=[
            pl.BlockSpec((1, N, 3), lambda i: (i, 0, 0)),
            pl.BlockSpec((1, 3, N), lambda i: (i, 0, 0)),
            pl.BlockSpec((1, S1, 3), lambda i: (i, 0, 0)),
            pl.BlockSpec((1, 3, S1), lambda i: (i, 0, 0)),
            pl.BlockSpec((1, S2, 3), lambda i: (i, 0, 0)),
        ] + full,
        out_specs=pl.BlockSpec((1, 1, 40), lambda i: (i, 0, 0)),
        out_shape=jax.ShapeDtypeStruct((B, 1, 40), jnp.float32),
    )(pos3, posT, c1T, rows1, c2T, *flat)
    return out.reshape(B, 40)


# speedup vs baseline: 1.1462x; 1.1462x over previous
import jax
import jax.numpy as jnp
from jax.experimental import pallas as pl
from jax.experimental.pallas import tpu as pltpu

B = 8
N = 1024
S1 = 512
K1 = 32
R1SQ = 0.2 * 0.2
S2 = 128
K2 = 64
R2SQ = 0.4 * 0.4
G1 = 8
G2 = 8

_F32 = jnp.float32
_NEG_INF = float('-inf')
_POS_INF = float('inf')


def _mm(a, b):
    return jnp.dot(a, b, preferred_element_type=jnp.float32)


def _tree(parts, op):
    while len(parts) > 1:
        nxt = [op(parts[i], parts[i + 1]) for i in range(0, len(parts) - 1, 2)]
        if len(parts) % 2:
            nxt.append(parts[-1])
        parts = nxt
    return parts[0]


def _row_reduce(x, op, jop):
    n = x.shape[1]
    parts = [x[:, k * 128:(k + 1) * 128] for k in range(n // 128)]
    return jop(_tree(parts, op), axis=1, keepdims=True)


def _fps_level(pts, S):
    px, py, pz = pts
    n = px.shape[1]
    nb = n // 128
    ii = jax.lax.broadcasted_iota(jnp.int32, (B, n), 1)
    iiS = jax.lax.broadcasted_iota(jnp.int32, (B, S), 1)


    x0 = px[:, 0:1]
    y0 = py[:, 0:1]
    z0 = pz[:, 0:1]
    mind = ((px - x0) * (px - x0) + (py - y0) * (py - y0)
            + (pz - z0) * (pz - z0))
    z = _F32(0.0)
    rowx = jnp.where(iiS == 0, x0, z)
    rowy = jnp.where(iiS == 0, y0, z)
    rowz = jnp.where(iiS == 0, z0, z)

    def body(i, carry):
        mind, rowx, rowy, rowz = carry
        mx = _row_reduce(mind, jnp.maximum, jnp.max)
        nxt = _row_reduce(jnp.where(mind == mx, ii, jnp.int32(n)),
                          jnp.minimum, jnp.min)
        oh = ii == nxt
        sx = _row_reduce(jnp.where(oh, px, z), jnp.add, jnp.sum)
        sy = _row_reduce(jnp.where(oh, py, z), jnp.add, jnp.sum)
        sz = _row_reduce(jnp.where(oh, pz, z), jnp.add, jnp.sum)
        ohS = iiS == i
        rowx = rowx + jnp.where(ohS, sx, z)
        rowy = rowy + jnp.where(ohS, sy, z)
        rowz = rowz + jnp.where(ohS, sz, z)
        d = ((px - sx) * (px - sx) + (py - sy) * (py - sy)
             + (pz - sz) * (pz - sz))
        return jnp.minimum(mind, d), rowx, rowy, rowz

    _, rowx, rowy, rowz = jax.lax.fori_loop(
        1, S, body, (mind, rowx, rowy, rowz), unroll=4)
    return rowx, rowy, rowz


def _fps_body(posT_ref, r1_ref, r2_ref):
    pT = posT_ref[:, :, :]
    px = pT[:, 0, :]
    py = pT[:, 1, :]
    pz = pT[:, 2, :]
    r1x, r1y, r1z = _fps_level((px, py, pz), S1)
    r1_ref[:, 0, :] = r1x
    r1_ref[:, 1, :] = r1y
    r1_ref[:, 2, :] = r1z
    r2x, r2y, r2z = _fps_level((r1x, r1y, r1z), S2)
    r2_ref[:, 0, :] = r2x
    r2_ref[:, 1, :] = r2y
    r2_ref[:, 2, :] = r2z


def _sa_body(pos3_ref, posT_ref, c1T_ref, r1_ref, c2T_ref,
             w11, b11, w12, b12, w13, b13,
             w21, b21, w22, b22, w23, b23,
             w31, b31, w32, b32, w33, b33,
             fw1, fb1, bg1, bb1, fw2, fb2, bg2, bb2, cw, cb,
             out_ref):
    p3 = pos3_ref[0]
    pT = posT_ref[0]
    px_row = pT[0:1, :]
    py_row = pT[1:2, :]
    pz_row = pT[2:3, :]
    c1 = c1T_ref[0]
    colx1 = c1[:, 0:1]
    coly1 = c1[:, 1:2]
    colz1 = c1[:, 2:3]
    r1 = r1_ref[0]
    r1x = r1[0:1, :]
    r1y = r1[1:2, :]
    r1z = r1[2:3, :]
    c2 = c2T_ref[0]
    colx2 = c2[:, 0:1]
    coly2 = c2[:, 1:2]
    colz2 = c2[:, 2:3]

    dx = colx1 - px_row
    dy = coly1 - py_row
    dz = colz1 - pz_row
    d2a = dx * dx + dy * dy + dz * dz
    jj1 = jax.lax.broadcasted_iota(jnp.int32, (S1, N), 1)

    def sa1_body(j, carry):
        d2m, hmax = carry
        rels = []
        keeps = []
        for _ in range(G1):
            m = jnp.min(d2m, axis=1, keepdims=True)
            nidx = jnp.min(jnp.where(d2m == m, jj1, jnp.int32(2 * N)),
                           axis=1, keepdims=True)
            oh = jj1 == nidx
            rels.append(_mm(oh.astype(jnp.float32), p3) - c1)
            keeps.append(m <= _F32(R1SQ))
            d2m = jnp.where(oh, _POS_INF, d2m)
        rel = jnp.concatenate(rels, axis=0)
        keep = jnp.concatenate(keeps, axis=0)
        a = jnp.maximum(_mm(rel, w11[:, :]) + b11[:, :], _F32(0.0))
        a = jnp.maximum(_mm(a, w12[:, :]) + b12[:, :], _F32(0.0))
        msg = jnp.maximum(_mm(a, w13[:, :]) + b13[:, :], _F32(0.0))
        msg = jnp.where(keep, msg, _NEG_INF)
        for s in range(G1):
            hmax = jnp.maximum(hmax, msg[s * S1:(s + 1) * S1, :])
        return d2m, hmax

    _, h1 = jax.lax.fori_loop(
        0, K1 // G1, sa1_body,
        (d2a, jnp.full((S1, 128), _NEG_INF, jnp.float32)), unroll=2)

    dx2 = colx2 - r1x
    dy2 = coly2 - r1y
    dz2 = colz2 - r1z
    d2b = dx2 * dx2 + dy2 * dy2 + dz2 * dz2
    jj2 = jax.lax.broadcasted_iota(jnp.int32, (S2, S1), 1)
    hp1 = jnp.concatenate([h1, c1], axis=1)

    def sa2_body(j, carry):
        d2m, hmax = carry
        gs = []
        keeps = []
        for _ in range(G2):
            m = jnp.min(d2m, axis=1, keepdims=True)
            nidx = jnp.min(jnp.where(d2m == m, jj2, jnp.int32(2 * S1)),
                           axis=1, keepdims=True)
            oh = jj2 == nidx
            gs.append(_mm(oh.astype(jnp.float32), hp1))
            keeps.append(m <= _F32(R2SQ))
            d2m = jnp.where(oh, _POS_INF, d2m)
        g = jnp.concatenate(gs, axis=0)
        rel3 = g[:, 128:131] - jnp.concatenate([c2] * G2, axis=0)
        keep = jnp.concatenate(keeps, axis=0)
        a = (_mm(g[:, 0:128], w21[0:128, :]) + _mm(rel3, w21[128:131, :])
             + b21[:, :])
        a = jnp.maximum(a, _F32(0.0))
        a = jnp.maximum(_mm(a, w22[:, :]) + b22[:, :], _F32(0.0))
        msg = jnp.maximum(_mm(a, w23[:, :]) + b23[:, :], _F32(0.0))
        msg = jnp.where(keep, msg, _NEG_INF)
        for s in range(G2):
            hmax = jnp.maximum(hmax, msg[s * S2:(s + 1) * S2, :])
        return d2m, hmax

    _, h2 = jax.lax.fori_loop(
        0, K2 // G2, sa2_body,
        (d2b, jnp.full((S2, 256), _NEG_INF, jnp.float32)), unroll=2)

    a = _mm(h2, w31[0:256, :]) + _mm(c2, w31[256:259, :]) + b31[:, :]
    a = jnp.maximum(a, _F32(0.0))
    a = jnp.maximum(_mm(a, w32[:, :]) + b32[:, :], _F32(0.0))
    a = jnp.maximum(_mm(a, w33[:, :]) + b33[:, :], _F32(0.0))
    g = jnp.max(a, axis=0, keepdims=True)

    inv = _F32(1.0) / jnp.sqrt(_F32(1.0 + 1e-5))
    x = _mm(g, fw1[:, :]) + fb1[:, :]
    x = jnp.maximum(x * inv * bg1[:, :] + bb1[:, :], _F32(0.0))
    x = _mm(x, fw2[:, :]) + fb2[:, :]
    x = jnp.maximum(x * inv * bg2[:, :] + bb2[:, :], _F32(0.0))
    out_ref[0] = _mm(x, cw[:, :]) + cb[:, :]


def kernel(pos, batch, params):
    del batch
    pos3 = pos.reshape(B, N, 3)
    posT = pos3.transpose(0, 2, 1)

    rows1, rows2 = pl.pallas_call(
        _fps_body,
        grid=(1,),
        in_specs=[pl.BlockSpec((B, 3, N), lambda i: (0, 0, 0))],
        out_specs=[
            pl.BlockSpec((B, 3, S1), lambda i: (0, 0, 0)),
            pl.BlockSpec((B, 3, S2), lambda i: (0, 0, 0)),
        ],
        out_shape=[
            jax.ShapeDtypeStruct((B, 3, S1), jnp.float32),
            jax.ShapeDtypeStruct((B, 3, S2), jnp.float32),
        ],
    )(posT)
    c1T = rows1.transpose(0, 2, 1)
    c2T = rows2.transpose(0, 2, 1)

    def row(v):
        return v.reshape(1, -1)

    flat = []
    for name in ('sa1', 'sa2', 'sa3'):
        for W, b in params[name]:
            flat += [W, row(b)]
    flat += [params['fc1'][0], row(params['fc1'][1]),
             row(params['bn1'][0]), row(params['bn1'][1]),
             params['fc2'][0], row(params['fc2'][1]),
             row(params['bn2'][0]), row(params['bn2'][1]),
             params['cls'][0], row(params['cls'][1])]
    full = [pl.BlockSpec(w.shape, lambda i, nd=w.ndim: (0,) * nd)
            for w in flat]

    out = pl.pallas_call(
        _sa_body,
        grid=(B,),
        in_specs=[
            pl.BlockSpec((1, N, 3), lambda i: (i, 0, 0)),
            pl.BlockSpec((1, 3, N), lambda i: (i, 0, 0)),
            pl.BlockSpec((1, S1, 3), lambda i: (i, 0, 0)),
            pl.BlockSpec((1, 3, S1), lambda i: (i, 0, 0)),
            pl.BlockSpec((1, S2, 3), lambda i: (i, 0, 0)),
        ] + full,
        out_specs=pl.BlockSpec((1, 1, 40), lambda i: (i, 0, 0)),
        out_shape=jax.ShapeDtypeStruct((B, 1, 40), jnp.float32),
    )(pos3, posT, c1T, rows1, c2T, *flat)
    return out.reshape(B, 40)


# speedup vs baseline: 1.1890x; 1.0373x over previous
import jax
import jax.numpy as jnp
from jax.experimental import pallas as pl
from jax.experimental.pallas import tpu as pltpu

B = 8
N = 1024
S1 = 512
K1 = 32
R1SQ = 0.2 * 0.2
S2 = 128
K2 = 64
R2SQ = 0.4 * 0.4
G1 = 8
G2 = 8

_F32 = jnp.float32
_NEG_INF = float('-inf')
_POS_INF = float('inf')


def _mm(a, b):
    return jnp.dot(a, b, preferred_element_type=jnp.float32)


def _tree(parts, op):
    while len(parts) > 1:
        nxt = [op(parts[i], parts[i + 1]) for i in range(0, len(parts) - 1, 2)]
        if len(parts) % 2:
            nxt.append(parts[-1])
        parts = nxt
    return parts[0]


def _row_reduce(x, op, jop):
    n = x.shape[1]
    parts = [x[:, k * 128:(k + 1) * 128] for k in range(n // 128)]
    return jop(_tree(parts, op), axis=1, keepdims=True)


def _fps_level(pts, S):
    px, py, pz = pts
    n = px.shape[1]
    nb = n // 128
    ii = jax.lax.broadcasted_iota(jnp.int32, (B, n), 1)
    iiS = jax.lax.broadcasted_iota(jnp.int32, (B, S), 1)


    x0 = px[:, 0:1]
    y0 = py[:, 0:1]
    z0 = pz[:, 0:1]
    mind = ((px - x0) * (px - x0) + (py - y0) * (py - y0)
            + (pz - z0) * (pz - z0))
    z = _F32(0.0)
    rowx = jnp.where(iiS == 0, x0, z)
    rowy = jnp.where(iiS == 0, y0, z)
    rowz = jnp.where(iiS == 0, z0, z)

    def body(i, carry):
        mind, rowx, rowy, rowz = carry
        mx = _row_reduce(mind, jnp.maximum, jnp.max)
        nxt = _row_reduce(jnp.where(mind == mx, ii, jnp.int32(n)),
                          jnp.minimum, jnp.min)
        oh = ii == nxt
        sx = _row_reduce(jnp.where(oh, px, z), jnp.add, jnp.sum)
        sy = _row_reduce(jnp.where(oh, py, z), jnp.add, jnp.sum)
        sz = _row_reduce(jnp.where(oh, pz, z), jnp.add, jnp.sum)
        ohS = iiS == i
        rowx = rowx + jnp.where(ohS, sx, z)
        rowy = rowy + jnp.where(ohS, sy, z)
        rowz = rowz + jnp.where(ohS, sz, z)
        d = ((px - sx) * (px - sx) + (py - sy) * (py - sy)
             + (pz - sz) * (pz - sz))
        return jnp.minimum(mind, d), rowx, rowy, rowz

    _, rowx, rowy, rowz = jax.lax.fori_loop(
        1, S, body, (mind, rowx, rowy, rowz), unroll=4)
    return rowx, rowy, rowz


def _fps_body(posT_ref, r1_ref, r2_ref):
    pT = posT_ref[:, :, :]
    px = pT[:, 0, :]
    py = pT[:, 1, :]
    pz = pT[:, 2, :]
    r1x, r1y, r1z = _fps_level((px, py, pz), S1)
    r1_ref[:, 0, :] = r1x
    r1_ref[:, 1, :] = r1y
    r1_ref[:, 2, :] = r1z
    r2x, r2y, r2z = _fps_level((r1x, r1y, r1z), S2)
    r2_ref[:, 0, :] = r2x
    r2_ref[:, 1, :] = r2y
    r2_ref[:, 2, :] = r2z


def _sa_body(pos3_ref, posT_ref, c1T_ref, r1_ref, c2T_ref,
             w11, b11, w12, b12, w13, b13,
             w21, b21, w22, b22, w23, b23,
             w31, b31, w32, b32, w33, b33,
             fw1, fb1, bg1, bb1, fw2, fb2, bg2, bb2, cw, cb,
             out_ref):
    p3 = pos3_ref[0]
    pT = posT_ref[0]
    px_row = pT[0:1, :]
    py_row = pT[1:2, :]
    pz_row = pT[2:3, :]
    c1 = c1T_ref[0]
    colx1 = c1[:, 0:1]
    coly1 = c1[:, 1:2]
    colz1 = c1[:, 2:3]
    r1 = r1_ref[0]
    r1x = r1[0:1, :]
    r1y = r1[1:2, :]
    r1z = r1[2:3, :]
    c2 = c2T_ref[0]
    colx2 = c2[:, 0:1]
    coly2 = c2[:, 1:2]
    colz2 = c2[:, 2:3]

    dx = colx1 - px_row
    dy = coly1 - py_row
    dz = colz1 - pz_row
    d2a = dx * dx + dy * dy + dz * dz
    jj1 = jax.lax.broadcasted_iota(jnp.int32, (S1, N), 1)

    def sa1_body(j, carry):
        d2m, hmax = carry
        rels = []
        keeps = []
        for _ in range(G1):
            m = jnp.min(d2m, axis=1, keepdims=True)
            nidx = jnp.min(jnp.where(d2m == m, jj1, jnp.int32(2 * N)),
                           axis=1, keepdims=True)
            oh = jj1 == nidx
            rels.append(_mm(oh.astype(jnp.float32), p3) - c1)
            keeps.append(m <= _F32(R1SQ))
            d2m = jnp.where(oh, _POS_INF, d2m)
        rel = jnp.concatenate(rels, axis=0)
        keep = jnp.concatenate(keeps, axis=0)
        a = jnp.maximum(_mm(rel, w11[:, :]) + b11[:, :], _F32(0.0))
        a = jnp.maximum(_mm(a, w12[:, :]) + b12[:, :], _F32(0.0))
        msg = jnp.maximum(_mm(a, w13[:, :]) + b13[:, :], _F32(0.0))
        msg = jnp.where(keep, msg, _NEG_INF)
        for s in range(G1):
            hmax = jnp.maximum(hmax, msg[s * S1:(s + 1) * S1, :])
        return d2m, hmax

    _, h1 = jax.lax.fori_loop(
        0, K1 // G1, sa1_body,
        (d2a, jnp.full((S1, 128), _NEG_INF, jnp.float32)), unroll=4)

    dx2 = colx2 - r1x
    dy2 = coly2 - r1y
    dz2 = colz2 - r1z
    d2b = dx2 * dx2 + dy2 * dy2 + dz2 * dz2
    jj2 = jax.lax.broadcasted_iota(jnp.int32, (S2, S1), 1)
    hp1 = jnp.concatenate([h1, c1], axis=1)

    def sa2_body(j, carry):
        d2m, hmax = carry
        gs = []
        keeps = []
        for _ in range(G2):
            m = jnp.min(d2m, axis=1, keepdims=True)
            nidx = jnp.min(jnp.where(d2m == m, jj2, jnp.int32(2 * S1)),
                           axis=1, keepdims=True)
            oh = jj2 == nidx
            gs.append(_mm(oh.astype(jnp.float32), hp1))
            keeps.append(m <= _F32(R2SQ))
            d2m = jnp.where(oh, _POS_INF, d2m)
        g = jnp.concatenate(gs, axis=0)
        rel3 = g[:, 128:131] - jnp.concatenate([c2] * G2, axis=0)
        keep = jnp.concatenate(keeps, axis=0)
        a = (_mm(g[:, 0:128], w21[0:128, :]) + _mm(rel3, w21[128:131, :])
             + b21[:, :])
        a = jnp.maximum(a, _F32(0.0))
        a = jnp.maximum(_mm(a, w22[:, :]) + b22[:, :], _F32(0.0))
        msg = jnp.maximum(_mm(a, w23[:, :]) + b23[:, :], _F32(0.0))
        msg = jnp.where(keep, msg, _NEG_INF)
        for s in range(G2):
            hmax = jnp.maximum(hmax, msg[s * S2:(s + 1) * S2, :])
        return d2m, hmax

    _, h2 = jax.lax.fori_loop(
        0, K2 // G2, sa2_body,
        (d2b, jnp.full((S2, 256), _NEG_INF, jnp.float32)), unroll=4)

    a = _mm(h2, w31[0:256, :]) + _mm(c2, w31[256:259, :]) + b31[:, :]
    a = jnp.maximum(a, _F32(0.0))
    a = jnp.maximum(_mm(a, w32[:, :]) + b32[:, :], _F32(0.0))
    a = jnp.maximum(_mm(a, w33[:, :]) + b33[:, :], _F32(0.0))
    g = jnp.max(a, axis=0, keepdims=True)

    inv = _F32(1.0) / jnp.sqrt(_F32(1.0 + 1e-5))
    x = _mm(g, fw1[:, :]) + fb1[:, :]
    x = jnp.maximum(x * inv * bg1[:, :] + bb1[:, :], _F32(0.0))
    x = _mm(x, fw2[:, :]) + fb2[:, :]
    x = jnp.maximum(x * inv * bg2[:, :] + bb2[:, :], _F32(0.0))
    out_ref[0] = _mm(x, cw[:, :]) + cb[:, :]


def kernel(pos, batch, params):
    del batch
    pos3 = pos.reshape(B, N, 3)
    posT = pos3.transpose(0, 2, 1)

    rows1, rows2 = pl.pallas_call(
        _fps_body,
        grid=(1,),
        in_specs=[pl.BlockSpec((B, 3, N), lambda i: (0, 0, 0))],
        out_specs=[
            pl.BlockSpec((B, 3, S1), lambda i: (0, 0, 0)),
            pl.BlockSpec((B, 3, S2), lambda i: (0, 0, 0)),
        ],
        out_shape=[
            jax.ShapeDtypeStruct((B, 3, S1), jnp.float32),
            jax.ShapeDtypeStruct((B, 3, S2), jnp.float32),
        ],
    )(posT)
    c1T = rows1.transpose(0, 2, 1)
    c2T = rows2.transpose(0, 2, 1)

    def row(v):
        return v.reshape(1, -1)

    flat = []
    for name in ('sa1', 'sa2', 'sa3'):
        for W, b in params[name]:
            flat += [W, row(b)]
    flat += [params['fc1'][0], row(params['fc1'][1]),
             row(params['bn1'][0]), row(params['bn1'][1]),
             params['fc2'][0], row(params['fc2'][1]),
             row(params['bn2'][0]), row(params['bn2'][1]),
             params['cls'][0], row(params['cls'][1])]
    full = [pl.BlockSpec(w.shape, lambda i, nd=w.ndim: (0,) * nd)
            for w in flat]

    out = pl.pallas_call(
        _sa_body,
        grid=(B,),
        in_specs=[
            pl.BlockSpec((1, N, 3), lambda i: (i, 0, 0)),
            pl.BlockSpec((1, 3, N), lambda i: (i, 0, 0)),
            pl.BlockSpec((1, S1, 3), lambda i: (i, 0, 0)),
            pl.BlockSpec((1, 3, S1), lambda i: (i, 0, 0)),
            pl.BlockSpec((1, S2, 3), lambda i: (i, 0, 0)),
        ] + full,
        out_specs=pl.BlockSpec((1, 1, 40), lambda i: (i, 0, 0)),
        out_shape=jax.ShapeDtypeStruct((B, 1, 40), jnp.float32),
    )(pos3, posT, c1T, rows1, c2T, *flat)
    return out.reshape(B, 40)


# speedup vs baseline: 1.1899x; 1.0007x over previous
import jax
import jax.numpy as jnp
from jax.experimental import pallas as pl

B = 8
N = 1024
S1 = 512
K1 = 32
R1SQ = 0.2 * 0.2
S2 = 128
K2 = 64
R2SQ = 0.4 * 0.4
G1 = 8
G2 = 8

_F32 = jnp.float32
_NEG_INF = float('-inf')
_POS_INF = float('inf')


def _mm(a, b):
    return jnp.dot(a, b, preferred_element_type=jnp.float32)


def _tree(parts, op):
    while len(parts) > 1:
        nxt = [op(parts[i], parts[i + 1]) for i in range(0, len(parts) - 1, 2)]
        if len(parts) % 2:
            nxt.append(parts[-1])
        parts = nxt
    return parts[0]


def _row_reduce(x, op, jop):
    n = x.shape[1]
    parts = [x[:, k * 128:(k + 1) * 128] for k in range(n // 128)]
    return jop(_tree(parts, op), axis=1, keepdims=True)


def _fps_level(pts, S):
    px, py, pz = pts
    n = px.shape[1]
    ii = jax.lax.broadcasted_iota(jnp.int32, (B, n), 1)
    iiS = jax.lax.broadcasted_iota(jnp.int32, (B, S), 1)

    x0 = px[:, 0:1]
    y0 = py[:, 0:1]
    z0 = pz[:, 0:1]
    mind = ((px - x0) * (px - x0) + (py - y0) * (py - y0)
            + (pz - z0) * (pz - z0))
    z = _F32(0.0)
    rowx = jnp.where(iiS == 0, x0, z)
    rowy = jnp.where(iiS == 0, y0, z)
    rowz = jnp.where(iiS == 0, z0, z)

    def body(i, carry):
        mind, rowx, rowy, rowz = carry
        mx = _row_reduce(mind, jnp.maximum, jnp.max)
        nxt = _row_reduce(jnp.where(mind == mx, ii, jnp.int32(n)),
                          jnp.minimum, jnp.min)
        oh = ii == nxt
        sx = _row_reduce(jnp.where(oh, px, z), jnp.add, jnp.sum)
        sy = _row_reduce(jnp.where(oh, py, z), jnp.add, jnp.sum)
        sz = _row_reduce(jnp.where(oh, pz, z), jnp.add, jnp.sum)
        ohS = iiS == i
        rowx = rowx + jnp.where(ohS, sx, z)
        rowy = rowy + jnp.where(ohS, sy, z)
        rowz = rowz + jnp.where(ohS, sz, z)
        d = ((px - sx) * (px - sx) + (py - sy) * (py - sy)
             + (pz - sz) * (pz - sz))
        return jnp.minimum(mind, d), rowx, rowy, rowz

    _, rowx, rowy, rowz = jax.lax.fori_loop(
        1, S, body, (mind, rowx, rowy, rowz), unroll=4)
    return rowx, rowy, rowz


def _fps_body(posT_ref, r1_ref, r2_ref):
    pT = posT_ref[:, :, :]
    px = pT[:, 0, :]
    py = pT[:, 1, :]
    pz = pT[:, 2, :]
    r1x, r1y, r1z = _fps_level((px, py, pz), S1)
    r1_ref[:, 0, :] = r1x
    r1_ref[:, 1, :] = r1y
    r1_ref[:, 2, :] = r1z
    r2x, r2y, r2z = _fps_level((r1x, r1y, r1z), S2)
    r2_ref[:, 0, :] = r2x
    r2_ref[:, 1, :] = r2y
    r2_ref[:, 2, :] = r2z


def _sa_body(pos3_ref, posT_ref, c1T_ref, r1_ref, c2T_ref,
             w11, b11, w12, b12, w13, b13,
             w21, b21, w22, b22, w23, b23,
             w31, b31, w32, b32, w33, b33,
             fw1, fb1, bg1, bb1, fw2, fb2, bg2, bb2, cw, cb,
             out_ref):
    p3 = pos3_ref[0]
    pT = posT_ref[0]
    px_row = pT[0:1, :]
    py_row = pT[1:2, :]
    pz_row = pT[2:3, :]
    c1 = c1T_ref[0]
    colx1 = c1[:, 0:1]
    coly1 = c1[:, 1:2]
    colz1 = c1[:, 2:3]
    r1 = r1_ref[0]
    r1x = r1[0:1, :]
    r1y = r1[1:2, :]
    r1z = r1[2:3, :]
    c2 = c2T_ref[0]
    colx2 = c2[:, 0:1]
    coly2 = c2[:, 1:2]
    colz2 = c2[:, 2:3]

    dx = colx1 - px_row
    dy = coly1 - py_row
    dz = colz1 - pz_row
    d2a = dx * dx + dy * dy + dz * dz
    jj1 = jax.lax.broadcasted_iota(jnp.int32, (S1, N), 1)

    def sa1_body(j, carry):
        d2m, hmax = carry
        rels = []
        keeps = []
        for _ in range(G1):
            m = jnp.min(d2m, axis=1, keepdims=True)
            nidx = jnp.min(jnp.where(d2m == m, jj1, jnp.int32(2 * N)),
                           axis=1, keepdims=True)
            oh = jj1 == nidx
            rels.append(_mm(oh.astype(jnp.float32), p3) - c1)
            keeps.append(m <= _F32(R1SQ))
            d2m = jnp.where(oh, _POS_INF, d2m)
        rel = jnp.concatenate(rels, axis=0)
        keep = jnp.concatenate(keeps, axis=0)
        a = jnp.maximum(_mm(rel, w11[:, :]) + b11[:, :], _F32(0.0))
        a = jnp.maximum(_mm(a, w12[:, :]) + b12[:, :], _F32(0.0))
        msg = jnp.maximum(_mm(a, w13[:, :]) + b13[:, :], _F32(0.0))
        msg = jnp.where(keep, msg, _NEG_INF)
        for s in range(G1):
            hmax = jnp.maximum(hmax, msg[s * S1:(s + 1) * S1, :])
        return d2m, hmax

    _, h1 = jax.lax.fori_loop(
        0, K1 // G1, sa1_body,
        (d2a, jnp.full((S1, 128), _NEG_INF, jnp.float32)), unroll=4)

    dx2 = colx2 - r1x
    dy2 = coly2 - r1y
    dz2 = colz2 - r1z
    d2b = dx2 * dx2 + dy2 * dy2 + dz2 * dz2
    jj2 = jax.lax.broadcasted_iota(jnp.int32, (S2, S1), 1)
    hp1 = jnp.concatenate([h1, c1], axis=1)

    def sa2_body(j, carry):
        d2m, hmax = carry
        gs = []
        keeps = []
        for _ in range(G2):
            m = jnp.min(d2m, axis=1, keepdims=True)
            nidx = jnp.min(jnp.where(d2m == m, jj2, jnp.int32(2 * S1)),
                           axis=1, keepdims=True)
            oh = jj2 == nidx
            gs.append(_mm(oh.astype(jnp.float32), hp1))
            keeps.append(m <= _F32(R2SQ))
            d2m = jnp.where(oh, _POS_INF, d2m)
        g = jnp.concatenate(gs, axis=0)
        rel3 = g[:, 128:131] - jnp.concatenate([c2] * G2, axis=0)
        keep = jnp.concatenate(keeps, axis=0)
        a = (_mm(g[:, 0:128], w21[0:128, :]) + _mm(rel3, w21[128:131, :])
             + b21[:, :])
        a = jnp.maximum(a, _F32(0.0))
        a = jnp.maximum(_mm(a, w22[:, :]) + b22[:, :], _F32(0.0))
        msg = jnp.maximum(_mm(a, w23[:, :]) + b23[:, :], _F32(0.0))
        msg = jnp.where(keep, msg, _NEG_INF)
        for s in range(G2):
            hmax = jnp.maximum(hmax, msg[s * S2:(s + 1) * S2, :])
        return d2m, hmax

    _, h2 = jax.lax.fori_loop(
        0, K2 // G2, sa2_body,
        (d2b, jnp.full((S2, 256), _NEG_INF, jnp.float32)), unroll=4)

    a = _mm(h2, w31[0:256, :]) + _mm(c2, w31[256:259, :]) + b31[:, :]
    a = jnp.maximum(a, _F32(0.0))
    a = jnp.maximum(_mm(a, w32[:, :]) + b32[:, :], _F32(0.0))
    a = jnp.maximum(_mm(a, w33[:, :]) + b33[:, :], _F32(0.0))
    g = jnp.max(a, axis=0, keepdims=True)

    inv = _F32(1.0) / jnp.sqrt(_F32(1.0 + 1e-5))
    x = _mm(g, fw1[:, :]) + fb1[:, :]
    x = jnp.maximum(x * inv * bg1[:, :] + bb1[:, :], _F32(0.0))
    x = _mm(x, fw2[:, :]) + fb2[:, :]
    x = jnp.maximum(x * inv * bg2[:, :] + bb2[:, :], _F32(0.0))
    out_ref[0] = _mm(x, cw[:, :]) + cb[:, :]


def kernel(pos, batch, params):
    del batch
    pos3 = pos.reshape(B, N, 3)
    posT = pos3.transpose(0, 2, 1)

    rows1, rows2 = pl.pallas_call(
        _fps_body,
        grid=(1,),
        in_specs=[pl.BlockSpec((B, 3, N), lambda i: (0, 0, 0))],
        out_specs=[
            pl.BlockSpec((B, 3, S1), lambda i: (0, 0, 0)),
            pl.BlockSpec((B, 3, S2), lambda i: (0, 0, 0)),
        ],
        out_shape=[
            jax.ShapeDtypeStruct((B, 3, S1), jnp.float32),
            jax.ShapeDtypeStruct((B, 3, S2), jnp.float32),
        ],
    )(posT)
    c1T = rows1.transpose(0, 2, 1)
    c2T = rows2.transpose(0, 2, 1)

    def row(v):
        return v.reshape(1, -1)

    flat = []
    for name in ('sa1', 'sa2', 'sa3'):
        for W, b in params[name]:
            flat += [W, row(b)]
    flat += [params['fc1'][0], row(params['fc1'][1]),
             row(params['bn1'][0]), row(params['bn1'][1]),
             params['fc2'][0], row(params['fc2'][1]),
             row(params['bn2'][0]), row(params['bn2'][1]),
             params['cls'][0], row(params['cls'][1])]
    full = [pl.BlockSpec(w.shape, lambda i, nd=w.ndim: (0,) * nd)
            for w in flat]

    out = pl.pallas_call(
        _sa_body,
        grid=(B,),
        in_specs=[
            pl.BlockSpec((1, N, 3), lambda i: (i, 0, 0)),
            pl.BlockSpec((1, 3, N), lambda i: (i, 0, 0)),
            pl.BlockSpec((1, S1, 3), lambda i: (i, 0, 0)),
            pl.BlockSpec((1, 3, S1), lambda i: (i, 0, 0)),
            pl.BlockSpec((1, S2, 3), lambda i: (i, 0, 0)),
        ] + full,
        out_specs=pl.BlockSpec((1, 1, 40), lambda i: (i, 0, 0)),
        out_shape=jax.ShapeDtypeStruct((B, 1, 40), jnp.float32),
    )(pos3, posT, c1T, rows1, c2T, *flat)
    return out.reshape(B, 40)
